# Initial kernel scaffold; baseline (speedup 1.0000x reference)
import jax
import jax.numpy as jnp
from jax.experimental import pallas as pl


def kernel(x, pos_x_w, pos_y_w, value_w, classify_w):
    raise NotImplementedError("write your pallas kernel here")


# speedup vs baseline: 8.6451x; 8.6451x over previous
import functools

import jax
import jax.numpy as jnp
from jax import lax
from jax.experimental import pallas as pl
from jax.experimental.pallas import tpu as pltpu
from jax.experimental.pallas import tpu_sc as plsc

LANES = 16


def _quantize(xv, n_levels):
    r = xv * jnp.float32(n_levels - 1)
    rh = r + jnp.float32(0.5)
    n = rh.astype(jnp.int32)
    tie = n.astype(jnp.float32) == rh
    odd = lax.rem(n, 2) == 1
    n = jnp.where(tie & odd, n - 1, n)
    return jnp.clip(n, 0, n_levels - 1)


def _sc_histograms(xf, size, n_levels, num_workers):
    B, S = xf.shape
    H = 2 * n_levels * size
    assert S % LANES == 0 and H % LANES == 0 and B % num_workers == 0
    b_per_w = B // num_workers
    n_chunks = S // LANES
    row_halves = (size + LANES - 1) // LANES

    mesh = plsc.VectorSubcoreMesh(core_axis_name="c", subcore_axis_name="s")

    @functools.partial(
        pl.kernel,
        out_type=jax.ShapeDtypeStruct((B * H,), jnp.float32),
        mesh=mesh,
        compiler_params=pltpu.CompilerParams(needs_layout_passes=False),
        scratch_types=[
            pltpu.VMEM((S,), jnp.float32),
            pltpu.VMEM((S,), jnp.int32),
            pltpu.VMEM((H,), jnp.float32),
        ],
    )
    def sc_kernel(x_hbm, out_hbm, xrow_v, lev_v, hist_v):
        wid = lax.axis_index("s") * 2 + lax.axis_index("c")
        iota = lax.iota(jnp.int32, LANES)
        ones = jnp.ones((LANES,), jnp.float32)
        for bi in range(b_per_w):
            b = wid * b_per_w + bi
            pltpu.sync_copy(x_hbm.at[pl.ds(b * S, S)], xrow_v)
            zeros = jnp.zeros((LANES,), jnp.float32)
            for c in range(H // LANES):
                hist_v[pl.ds(c * LANES, LANES)] = zeros
            for k in range(n_chunks):
                xv = xrow_v[pl.ds(k * LANES, LANES)]
                lev = _quantize(xv, n_levels)
                lev_v[pl.ds(k * LANES, LANES)] = lev
                col = lax.rem(k * LANES + iota, size)
                plsc.addupdate_scatter(hist_v, [lev * size + col], ones)
            for j in range(size):
                for h in range(row_halves):
                    i = h * LANES + iota
                    m = i < size
                    lev = plsc.load_gather(lev_v, [i * size + j], mask=m)
                    plsc.addupdate_scatter(
                        hist_v, [n_levels * size + lev * size + i], ones,
                        mask=m)
            pltpu.sync_copy(hist_v, out_hbm.at[pl.ds(b * H, H)])

    return sc_kernel(xf.reshape(-1)).reshape(B, H)


def _tc_combine(C, pos_x_w, pos_y_w, value_w, classify_wT):
    B = C.shape[0]
    size, D = pos_x_w.shape
    L = value_w.shape[0]
    n_cls = classify_wT.shape[1]

    def body(c_ref, px_ref, py_ref, vw_ref, cwt_ref, out_ref):
        parts = []
        for l in range(L):
            parts.append(vw_ref[l:l + 1, :] * px_ref[...])
        for l in range(L):
            parts.append(vw_ref[l:l + 1, :] * py_ref[...])
        U = jnp.concatenate(parts, axis=0)
        multiset = lax.dot_general(
            c_ref[...], U, (((1,), (0,)), ((), ())),
            preferred_element_type=jnp.float32)
        enc = jnp.where(multiset > 0, jnp.float32(1.0), jnp.float32(-1.0))
        out_ref[...] = lax.dot_general(
            enc, cwt_ref[...], (((1,), (0,)), ((), ())),
            precision=lax.Precision.HIGHEST,
            preferred_element_type=jnp.float32)

    return pl.pallas_call(
        body,
        out_shape=jax.ShapeDtypeStruct((B, n_cls), jnp.float32),
    )(C, pos_x_w, pos_y_w, value_w, classify_wT)


def kernel(x, pos_x_w, pos_y_w, value_w, classify_w):
    B = x.shape[0]
    size = pos_x_w.shape[0]
    n_levels = value_w.shape[0]
    xf = x.reshape(B, size * size)
    info = plsc.get_sparse_core_info()
    num_workers = info.num_cores * info.num_subcores
    C = _sc_histograms(xf, size, n_levels, num_workers)
    return _tc_combine(C, pos_x_w, pos_y_w, value_w, classify_w.T)


# speedup vs baseline: 10.9066x; 1.2616x over previous
import functools

import jax
import jax.numpy as jnp
from jax import lax
from jax.experimental import pallas as pl
from jax.experimental.pallas import tpu as pltpu
from jax.experimental.pallas import tpu_sc as plsc

LANES = 16


def _quantize(xv, n_levels):
    r = xv * jnp.float32(n_levels - 1)
    rh = r + jnp.float32(0.5)
    n = rh.astype(jnp.int32)
    tie = n.astype(jnp.float32) == rh
    odd = lax.rem(n, 2) == 1
    n = jnp.where(tie & odd, n - 1, n)
    return jnp.clip(n, 0, n_levels - 1)


def _sc_histograms(xf, size, n_levels, num_workers):
    B, S = xf.shape
    H = 2 * n_levels * size
    assert S % LANES == 0 and H % LANES == 0 and B % num_workers == 0
    b_per_w = B // num_workers
    n_chunks = S // LANES
    row_halves = (size + LANES - 1) // LANES

    mesh = plsc.VectorSubcoreMesh(core_axis_name="c", subcore_axis_name="s")

    @functools.partial(
        pl.kernel,
        out_type=jax.ShapeDtypeStruct((B, H), jnp.float32),
        mesh=mesh,
        compiler_params=pltpu.CompilerParams(needs_layout_passes=False),
        scratch_types=[
            pltpu.VMEM((S,), jnp.float32),
            pltpu.VMEM((S,), jnp.int32),
            pltpu.VMEM((H,), jnp.float32),
        ],
    )
    def sc_kernel(x_hbm, out_hbm, xrow_v, lev_v, hist_v):
        wid = lax.axis_index("s") * 2 + lax.axis_index("c")
        iota = lax.iota(jnp.int32, LANES)
        ones = jnp.ones((LANES,), jnp.float32)
        zeros = jnp.zeros((LANES,), jnp.float32)
        for bi in range(b_per_w):
            b = wid * b_per_w + bi
            pltpu.sync_copy(x_hbm.at[pl.ds(b * S, S)], xrow_v)

            @plsc.parallel_loop(0, H // LANES)
            def _zero(c):
                hist_v[pl.ds(c * LANES, LANES)] = zeros

            @plsc.parallel_loop(0, n_chunks)
            def _pass1(k):
                xv = xrow_v[pl.ds(k * LANES, LANES)]
                lev = _quantize(xv, n_levels)
                lev_v[pl.ds(k * LANES, LANES)] = lev
                col = lax.rem(k * LANES + iota, size)
                plsc.addupdate_scatter(hist_v, [lev * size + col], ones)

            @plsc.parallel_loop(0, size * row_halves)
            def _pass2(t):
                j = t // row_halves
                h = lax.rem(t, row_halves)
                i = h * LANES + iota
                m = i < size
                lev = plsc.load_gather(lev_v, [i * size + j], mask=m)
                plsc.addupdate_scatter(
                    hist_v, [n_levels * size + lev * size + i], ones,
                    mask=m)

            pltpu.sync_copy(hist_v, out_hbm.at[b])

    return sc_kernel(xf.reshape(-1))


def _tc_combine(C, pos_x_w, pos_y_w, value_w, classify_wT):
    B = C.shape[0]
    size, D = pos_x_w.shape
    L = value_w.shape[0]
    n_cls = classify_wT.shape[1]

    def body(c_ref, px_ref, py_ref, vw_ref, cwt_ref, out_ref):
        parts = []
        for l in range(L):
            parts.append(vw_ref[l:l + 1, :] * px_ref[...])
        for l in range(L):
            parts.append(vw_ref[l:l + 1, :] * py_ref[...])
        U = jnp.concatenate(parts, axis=0)
        multiset = lax.dot_general(
            c_ref[...], U, (((1,), (0,)), ((), ())),
            preferred_element_type=jnp.float32)
        enc = jnp.where(multiset > 0, jnp.float32(1.0), jnp.float32(-1.0))
        out_ref[...] = lax.dot_general(
            enc, cwt_ref[...], (((1,), (0,)), ((), ())),
            precision=lax.Precision.HIGHEST,
            preferred_element_type=jnp.float32)

    return pl.pallas_call(
        body,
        out_shape=jax.ShapeDtypeStruct((B, n_cls), jnp.float32),
    )(C, pos_x_w, pos_y_w, value_w, classify_wT)


def kernel(x, pos_x_w, pos_y_w, value_w, classify_w):
    B = x.shape[0]
    size = pos_x_w.shape[0]
    n_levels = value_w.shape[0]
    xf = x.reshape(B, size * size)
    info = plsc.get_sparse_core_info()
    num_workers = info.num_cores * info.num_subcores
    C = _sc_histograms(xf, size, n_levels, num_workers)
    return _tc_combine(C, pos_x_w, pos_y_w, value_w, classify_w.T)
